# baseline (speedup 1.0000x reference)
import functools

import jax
import jax.numpy as jnp
from jax import lax
from jax.experimental import pallas as pl
from jax.experimental.pallas import tpu as pltpu
from jax.experimental.pallas import tpu_sc as plsc

_INFO = plsc.get_sparse_core_info()
_NC = _INFO.num_cores
_NS = _INFO.num_subcores
_L = _INFO.num_lanes
_NW = _NC * _NS


@functools.partial(jax.jit, static_argnums=(1,))
def _table_lookup(param, B):
    chunk = B // _NW
    n_vecs = chunk // _L
    mesh = plsc.VectorSubcoreMesh(core_axis_name="c", subcore_axis_name="s")

    @functools.partial(
        pl.kernel,
        mesh=mesh,
        out_type=(
            jax.ShapeDtypeStruct((B,), jnp.float32),
            jax.ShapeDtypeStruct((B,), jnp.float32),
        ),
        scratch_types=[
            pltpu.VMEM((chunk,), jnp.int32),
            pltpu.VMEM((chunk,), jnp.float32),
            pltpu.VMEM((chunk,), jnp.float32),
            pltpu.SemaphoreType.DMA,
            pltpu.SemaphoreType.DMA,
        ],
    )
    def k(param_hbm, out0_hbm, out1_hbm, idx_v, rows0, rows1, sem0, sem1):
        wid = lax.axis_index("s") * _NC + lax.axis_index("c")
        base = wid * chunk

        zeros = jnp.zeros((_L,), jnp.int32)

        def fill(i, carry):
            idx_v[pl.ds(i * _L, _L)] = zeros
            return carry

        lax.fori_loop(0, n_vecs, fill, 0)
        cp0 = pltpu.async_copy(param_hbm.at[idx_v], rows0, sem0)

        def fill1(i, carry):
            idx_v[pl.ds(i * _L, _L)] = zeros + 1
            return carry

        cp0.wait()
        lax.fori_loop(0, n_vecs, fill1, 0)
        cp1 = pltpu.async_copy(param_hbm.at[idx_v], rows1, sem1)
        pltpu.sync_copy(rows0, out0_hbm.at[pl.ds(base, chunk)])
        cp1.wait()
        pltpu.sync_copy(rows1, out1_hbm.at[pl.ds(base, chunk)])

    return k(param)


def kernel(x, x_pa, param):
    B = x.shape[0]
    out0, out1 = _table_lookup(param, B)
    return (out0[:, None], out1[:, None])


# speedup vs baseline: 7.9249x; 7.9249x over previous
import functools

import jax
import jax.numpy as jnp
from jax import lax
from jax.experimental import pallas as pl
from jax.experimental.pallas import tpu as pltpu
from jax.experimental.pallas import tpu_sc as plsc

_INFO = plsc.get_sparse_core_info()
_NC = _INFO.num_cores
_NS = _INFO.num_subcores
_L = _INFO.num_lanes
_NW = _NC * _NS


@functools.partial(jax.jit, static_argnums=(1,))
def _table_broadcast(param_rep, B):
    chunk = B // _NW
    n_vecs = chunk // _L
    mesh = plsc.VectorSubcoreMesh(core_axis_name="c", subcore_axis_name="s")

    @functools.partial(
        pl.kernel,
        mesh=mesh,
        out_type=(
            jax.ShapeDtypeStruct((B,), jnp.float32),
            jax.ShapeDtypeStruct((B,), jnp.float32),
        ),
        scratch_types=[
            pltpu.VMEM((2 * _L,), jnp.float32),
            pltpu.VMEM((chunk,), jnp.float32),
            pltpu.VMEM((chunk,), jnp.float32),
        ],
    )
    def k(rep_hbm, out0_hbm, out1_hbm, rep_v, buf0, buf1):
        wid = lax.axis_index("s") * _NC + lax.axis_index("c")
        base = wid * chunk
        pltpu.sync_copy(rep_hbm, rep_v)
        vec0 = rep_v[pl.ds(0, _L)]
        vec1 = rep_v[pl.ds(_L, _L)]

        def fill(i, carry):
            buf0[pl.ds(i * _L, _L)] = vec0
            buf1[pl.ds(i * _L, _L)] = vec1
            return carry

        lax.fori_loop(0, n_vecs, fill, 0)
        pltpu.sync_copy(buf0, out0_hbm.at[pl.ds(base, chunk)])
        pltpu.sync_copy(buf1, out1_hbm.at[pl.ds(base, chunk)])

    return k(param_rep)


def kernel(x, x_pa, param):
    B = x.shape[0]
    param_rep = jnp.repeat(param, _L)
    out0, out1 = _table_broadcast(param_rep, B)
    return (out0[:, None], out1[:, None])


# speedup vs baseline: 92.7292x; 11.7010x over previous
import jax
import jax.numpy as jnp
from jax.experimental import pallas as pl
from jax.experimental.pallas import tpu as pltpu


def _broadcast_body(param_ref, out0_ref, out1_ref):
    out0_ref[...] = jnp.full(out0_ref.shape, param_ref[0], jnp.float32)
    out1_ref[...] = jnp.full(out1_ref.shape, param_ref[1], jnp.float32)


def kernel(x, x_pa, param):
    B = x.shape[0]
    rows = B // 128
    out0, out1 = pl.pallas_call(
        _broadcast_body,
        in_specs=[pl.BlockSpec(memory_space=pltpu.SMEM)],
        out_shape=(
            jax.ShapeDtypeStruct((rows, 128), jnp.float32),
            jax.ShapeDtypeStruct((rows, 128), jnp.float32),
        ),
    )(param)
    return (out0.reshape(B, 1), out1.reshape(B, 1))
